# baseline (speedup 1.0000x reference)
import jax
import jax.numpy as jnp
from jax import lax
from jax.experimental import pallas as pl
from jax.experimental.pallas import tpu as pltpu
from jax.experimental.pallas import tpu_sc as plsc

K = 1024
D = 64
BETA = 0.25
B = 16
T = 1024
N = B * T

NUM_WORKERS = 32
ROWS_PER_WORKER = N // NUM_WORKERS
CHUNK = 64


def _argmin_body(x_ref, w_ref, a_ref, bsq_ref, idx_ref):
    x = x_ref[0]
    w = w_ref[...]
    a = a_ref[0]
    bsq = bsq_ref[...]

    m2 = lax.dot_general(w + w, x, (((1,), (0,)), ((), ())),
                         preferred_element_type=jnp.float32)
    dist = (a + bsq) - m2

    minv = jnp.min(dist, axis=0, keepdims=True)
    iota_k = lax.broadcasted_iota(jnp.int32, (K, T), 0)
    idx = jnp.min(jnp.where(dist <= minv, iota_k, K), axis=0,
                  keepdims=True)
    idx_ref[0] = idx


NPARTS = 4
BP = B // NPARTS


def _argmin_call(inputs, w, a3, bsq, part):
    return pl.pallas_call(
        _argmin_body,
        grid=(BP,),
        in_specs=[
            pl.BlockSpec((1, D, T), lambda b: (part * BP + b, 0, 0)),
            pl.BlockSpec((K, D), lambda b: (0, 0)),
            pl.BlockSpec((1, 1, T), lambda b: (part * BP + b, 0, 0)),
            pl.BlockSpec((K, 1), lambda b: (0, 0)),
        ],
        out_specs=pl.BlockSpec((1, 1, T), lambda b: (b, 0, 0)),
        out_shape=jax.ShapeDtypeStruct((BP, 1, T), jnp.int32),
        compiler_params=pltpu.CompilerParams(
            dimension_semantics=("arbitrary",)),
    )(inputs, w, a3, bsq)


def _outputs_body(x_ref, w_ref, idx_ref, qst_ref, loss_ref, perp_ref,
                  lacc_ref, hist_ref):
    b = pl.program_id(0)

    @pl.when(b == 0)
    def _init():
        lacc_ref[0, 0] = 0.0
        hist_ref[...] = jnp.zeros_like(hist_ref)

    x = x_ref[0]
    w = w_ref[...]
    idx = idx_ref[0]

    iota_k = lax.broadcasted_iota(jnp.int32, (K, T), 0)
    enc_t = (iota_k == idx).astype(jnp.float32)
    hist_ref[...] += jnp.sum(enc_t, axis=1, keepdims=True)

    q = lax.dot_general(w, enc_t, (((0,), (0,)), ((), ())),
                        preferred_element_type=jnp.float32)
    diff = q - x
    qst_ref[0] = x + diff
    lacc_ref[0, 0] += jnp.sum(diff * diff)

    @pl.when(b == B - 1)
    def _fin():
        mean_sq = lacc_ref[0, 0] / (B * T * D)
        loss_ref[0, 0] = mean_sq + BETA * mean_sq
        avg = hist_ref[...] * (1.0 / N)
        ent = avg * jnp.log(avg + 1e-10)
        perp_ref[0, 0] = jnp.exp(-jnp.sum(ent))


def _outputs_call(inputs, w, idx):
    return pl.pallas_call(
        _outputs_body,
        grid=(B,),
        in_specs=[
            pl.BlockSpec((1, D, T), lambda b: (b, 0, 0)),
            pl.BlockSpec((K, D), lambda b: (0, 0)),
            pl.BlockSpec((1, 1, T), lambda b: (b, 0, 0)),
        ],
        out_specs=[
            pl.BlockSpec((1, D, T), lambda b: (b, 0, 0)),
            pl.BlockSpec(memory_space=pltpu.SMEM),
            pl.BlockSpec(memory_space=pltpu.SMEM),
        ],
        out_shape=[
            jax.ShapeDtypeStruct((B, D, T), jnp.float32),
            jax.ShapeDtypeStruct((1, 1), jnp.float32),
            jax.ShapeDtypeStruct((1, 1), jnp.float32),
        ],
        scratch_shapes=[
            pltpu.SMEM((1, 1), jnp.float32),
            pltpu.VMEM((K, 1), jnp.float32),
        ],
        compiler_params=pltpu.CompilerParams(
            dimension_semantics=("arbitrary",)),
    )(inputs, w, idx)


_SC_MESH = plsc.VectorSubcoreMesh(core_axis_name="c", subcore_axis_name="s")

NP = N // NPARTS
RPW = NP // NUM_WORKERS


def _sc_part_body(part, idx_hbm, zeros_hbm, out_ref, idx_v, rows_v):
    wid = lax.axis_index("s") * 2 + lax.axis_index("c")
    base = wid * RPW

    pltpu.sync_copy(idx_hbm.at[pl.ds(base, RPW)], idx_v)
    pltpu.sync_copy(zeros_hbm, rows_v)

    ones16 = jnp.full((16,), 1.0, jnp.float32)
    zero16 = jnp.zeros((16,), jnp.float32)
    lane = lax.iota(jnp.int32, 16)

    out_base = part * NP + base
    for ci in range(RPW // CHUNK):
        for g in range(CHUNK // 16):
            cols = idx_v[pl.ds(ci * CHUNK + g * 16, 16)]
            rows = lane + g * 16
            plsc.store_scatter(rows_v, [rows, cols], ones16)
        pltpu.sync_copy(rows_v,
                        out_ref.at[pl.ds(out_base + ci * CHUNK, CHUNK)])
        for g in range(CHUNK // 16):
            cols = idx_v[pl.ds(ci * CHUNK + g * 16, 16)]
            rows = lane + g * 16
            plsc.store_scatter(rows_v, [rows, cols], zero16)


def _sc_part(part, idx_part_flat, zeros_chunk, enc_ref):
    f = pl.kernel(
        lambda *args: _sc_part_body(part, *args),
        mesh=_SC_MESH,
        scratch_types=[
            pltpu.VMEM((RPW,), jnp.int32),
            pltpu.VMEM((CHUNK, K), jnp.float32),
        ],
        compiler_params=pltpu.CompilerParams(needs_layout_passes=False),
    )
    f(idx_part_flat, zeros_chunk, enc_ref)


@jax.jit
def kernel(inputs, W):
    flat = jnp.transpose(inputs, (0, 2, 1)).reshape(-1, D)
    a3 = jnp.sum(flat ** 2, axis=1).reshape(B, 1, T)
    bsq = jnp.sum(W ** 2, axis=1)[:, None]

    zeros_chunk = jnp.zeros((CHUNK, K), jnp.float32)
    enc_ref = jax.empty_ref(jax.ShapeDtypeStruct((N, K), jnp.float32))

    idx_parts = []
    for p in range(NPARTS):
        idx_p = _argmin_call(inputs, W, a3, bsq, p)
        _sc_part(p, idx_p.reshape(NP), zeros_chunk, enc_ref)
        idx_parts.append(idx_p)
    idx = jnp.concatenate(idx_parts, axis=0)

    qst, loss, perp = _outputs_call(inputs, W, idx)

    enc = enc_ref[...]
    return (loss.reshape(()), qst, perp.reshape(()), enc)


# speedup vs baseline: 1.2998x; 1.2998x over previous
import jax
import jax.numpy as jnp
from jax import lax
from jax.experimental import pallas as pl
from jax.experimental.pallas import tpu as pltpu
from jax.experimental.pallas import tpu_sc as plsc

K = 1024
D = 64
BETA = 0.25
B = 16
T = 1024
N = B * T

NUM_WORKERS = 32
ROWS_PER_WORKER = N // NUM_WORKERS
CHUNK = 64


def _argmin_body(x_ref, w_ref, a_ref, bsq_ref, idx_ref):
    b = pl.program_id(0)
    x = x_ref[0]
    w = w_ref[...]
    a = a_ref[b]
    bsq = bsq_ref[...]

    m2 = lax.dot_general(w + w, x, (((1,), (0,)), ((), ())),
                         preferred_element_type=jnp.float32)
    dist = (a + bsq) - m2

    minv = jnp.min(dist, axis=0, keepdims=True)
    iota_k = lax.broadcasted_iota(jnp.int32, (K, T), 0)
    idx = jnp.min(jnp.where(dist <= minv, iota_k, K), axis=0,
                  keepdims=True)
    idx_ref[b] = idx


def _argmin_call(inputs, w, a3, bsq):
    return pl.pallas_call(
        _argmin_body,
        grid=(B,),
        in_specs=[
            pl.BlockSpec((1, D, T), lambda b: (b, 0, 0)),
            pl.BlockSpec((K, D), lambda b: (0, 0)),
            pl.BlockSpec((B, 1, T), lambda b: (0, 0, 0)),
            pl.BlockSpec((K, 1), lambda b: (0, 0)),
        ],
        out_specs=pl.BlockSpec((B, 1, T), lambda b: (0, 0, 0)),
        out_shape=jax.ShapeDtypeStruct((B, 1, T), jnp.int32),
        compiler_params=pltpu.CompilerParams(
            dimension_semantics=("arbitrary",)),
    )(inputs, w, a3, bsq)


def _outputs_body(x_ref, w_ref, idx_ref, qst_ref, loss_ref, perp_ref,
                  lacc_ref, hist_ref):
    b = pl.program_id(0)

    @pl.when(b == 0)
    def _init():
        lacc_ref[0, 0] = 0.0
        hist_ref[...] = jnp.zeros_like(hist_ref)

    x = x_ref[0]
    w = w_ref[...]
    idx = idx_ref[b]

    iota_k = lax.broadcasted_iota(jnp.int32, (K, T), 0)
    enc_t = (iota_k == idx).astype(jnp.float32)
    hist_ref[...] += jnp.sum(enc_t, axis=1, keepdims=True)

    q = lax.dot_general(w, enc_t, (((0,), (0,)), ((), ())),
                        preferred_element_type=jnp.float32)
    diff = q - x
    qst_ref[0] = x + diff
    lacc_ref[0, 0] += jnp.sum(diff * diff)

    @pl.when(b == B - 1)
    def _fin():
        mean_sq = lacc_ref[0, 0] / (B * T * D)
        loss_ref[0, 0] = mean_sq + BETA * mean_sq
        avg = hist_ref[...] * (1.0 / N)
        ent = avg * jnp.log(avg + 1e-10)
        perp_ref[0, 0] = jnp.exp(-jnp.sum(ent))


def _outputs_call(inputs, w, idx):
    return pl.pallas_call(
        _outputs_body,
        grid=(B,),
        in_specs=[
            pl.BlockSpec((1, D, T), lambda b: (b, 0, 0)),
            pl.BlockSpec((K, D), lambda b: (0, 0)),
            pl.BlockSpec((B, 1, T), lambda b: (0, 0, 0)),
        ],
        out_specs=[
            pl.BlockSpec((1, D, T), lambda b: (b, 0, 0)),
            pl.BlockSpec(memory_space=pltpu.SMEM),
            pl.BlockSpec(memory_space=pltpu.SMEM),
        ],
        out_shape=[
            jax.ShapeDtypeStruct((B, D, T), jnp.float32),
            jax.ShapeDtypeStruct((1, 1), jnp.float32),
            jax.ShapeDtypeStruct((1, 1), jnp.float32),
        ],
        scratch_shapes=[
            pltpu.SMEM((1, 1), jnp.float32),
            pltpu.VMEM((K, 1), jnp.float32),
        ],
        compiler_params=pltpu.CompilerParams(
            dimension_semantics=("arbitrary",)),
    )(inputs, w, idx)


def _sc_scatter_body(idx_hbm, zeros_hbm, out_hbm, idx_v, rows_v):
    wid = lax.axis_index("s") * 2 + lax.axis_index("c")
    base = wid * ROWS_PER_WORKER

    pltpu.sync_copy(idx_hbm.at[pl.ds(base, ROWS_PER_WORKER)], idx_v)
    pltpu.sync_copy(zeros_hbm, rows_v)

    ones16 = jnp.full((16,), 1.0, jnp.float32)
    zero16 = jnp.zeros((16,), jnp.float32)
    lane = lax.iota(jnp.int32, 16)

    def chunk_body(ci, carry):
        for g in range(CHUNK // 16):
            cols = idx_v[pl.ds(ci * CHUNK + g * 16, 16)]
            rows = lane + g * 16
            plsc.store_scatter(rows_v, [rows, cols], ones16)
        pltpu.sync_copy(rows_v, out_hbm.at[pl.ds(base + ci * CHUNK, CHUNK)])
        for g in range(CHUNK // 16):
            cols = idx_v[pl.ds(ci * CHUNK + g * 16, 16)]
            rows = lane + g * 16
            plsc.store_scatter(rows_v, [rows, cols], zero16)
        return carry

    lax.fori_loop(0, ROWS_PER_WORKER // CHUNK, chunk_body, 0)


def _sc_scatter(idx_flat, zeros_chunk):
    mesh = plsc.VectorSubcoreMesh(core_axis_name="c", subcore_axis_name="s")
    f = pl.kernel(
        _sc_scatter_body,
        out_type=jax.ShapeDtypeStruct((N, K), jnp.float32),
        mesh=mesh,
        scratch_types=[
            pltpu.VMEM((ROWS_PER_WORKER,), jnp.int32),
            pltpu.VMEM((CHUNK, K), jnp.float32),
        ],
        compiler_params=pltpu.CompilerParams(needs_layout_passes=False),
    )
    return f(idx_flat, zeros_chunk)


@jax.jit
def kernel(inputs, W):
    flat = jnp.transpose(inputs, (0, 2, 1)).reshape(-1, D)
    a3 = jnp.sum(flat ** 2, axis=1).reshape(B, 1, T)
    bsq = jnp.sum(W ** 2, axis=1)[:, None]

    idx = _argmin_call(inputs, W, a3, bsq)

    zeros_chunk = jnp.zeros((CHUNK, K), jnp.float32)
    enc = _sc_scatter(idx.reshape(N), zeros_chunk)

    qst, loss, perp = _outputs_call(inputs, W, idx)

    return (loss.reshape(()), qst, perp.reshape(()), enc)


# speedup vs baseline: 1.4069x; 1.0824x over previous
import jax
import jax.numpy as jnp
from jax import lax
from jax.experimental import pallas as pl
from jax.experimental.pallas import tpu as pltpu
from jax.experimental.pallas import tpu_sc as plsc

K = 1024
D = 64
BETA = 0.25
B = 16
T = 1024
N = B * T

NUM_WORKERS = 32
ROWS_PER_WORKER = N // NUM_WORKERS
CHUNK = 64


def _argmin_body(x_ref, w_ref, idx_ref):
    b = pl.program_id(0)
    x = x_ref[0]
    w = w_ref[...]
    a = jnp.sum(x * x, axis=0, keepdims=True)
    bsq = jnp.sum(w * w, axis=1, keepdims=True)

    m2 = lax.dot_general(w + w, x, (((1,), (0,)), ((), ())),
                         preferred_element_type=jnp.float32)
    dist = (a + bsq) - m2

    minv = jnp.min(dist, axis=0, keepdims=True)
    iota_k = lax.broadcasted_iota(jnp.int32, (K, T), 0)
    idx = jnp.min(jnp.where(dist <= minv, iota_k, K), axis=0,
                  keepdims=True)
    idx_ref[b] = idx


def _argmin_call(inputs, w):
    return pl.pallas_call(
        _argmin_body,
        grid=(B,),
        in_specs=[
            pl.BlockSpec((1, D, T), lambda b: (b, 0, 0)),
            pl.BlockSpec((K, D), lambda b: (0, 0)),
        ],
        out_specs=pl.BlockSpec((B, 1, T), lambda b: (0, 0, 0)),
        out_shape=jax.ShapeDtypeStruct((B, 1, T), jnp.int32),
        compiler_params=pltpu.CompilerParams(
            dimension_semantics=("arbitrary",)),
    )(inputs, w)


def _outputs_body(x_ref, w_ref, idx_ref, qst_ref, loss_ref, perp_ref,
                  lacc_ref, hist_ref):
    b = pl.program_id(0)

    @pl.when(b == 0)
    def _init():
        lacc_ref[0, 0] = 0.0
        hist_ref[...] = jnp.zeros_like(hist_ref)

    x = x_ref[0]
    w = w_ref[...]
    idx = idx_ref[b]

    iota_k = lax.broadcasted_iota(jnp.int32, (K, T), 0)
    enc_t = (iota_k == idx).astype(jnp.float32)
    hist_ref[...] += jnp.sum(enc_t, axis=1, keepdims=True)

    q = lax.dot_general(w, enc_t, (((0,), (0,)), ((), ())),
                        preferred_element_type=jnp.float32)
    diff = q - x
    qst_ref[0] = x + diff
    lacc_ref[0, 0] += jnp.sum(diff * diff)

    @pl.when(b == B - 1)
    def _fin():
        mean_sq = lacc_ref[0, 0] / (B * T * D)
        loss_ref[0, 0] = mean_sq + BETA * mean_sq
        avg = hist_ref[...] * (1.0 / N)
        ent = avg * jnp.log(avg + 1e-10)
        perp_ref[0, 0] = jnp.exp(-jnp.sum(ent))


def _outputs_call(inputs, w, idx):
    return pl.pallas_call(
        _outputs_body,
        grid=(B,),
        in_specs=[
            pl.BlockSpec((1, D, T), lambda b: (b, 0, 0)),
            pl.BlockSpec((K, D), lambda b: (0, 0)),
            pl.BlockSpec((B, 1, T), lambda b: (0, 0, 0)),
        ],
        out_specs=[
            pl.BlockSpec((1, D, T), lambda b: (b, 0, 0)),
            pl.BlockSpec(memory_space=pltpu.SMEM),
            pl.BlockSpec(memory_space=pltpu.SMEM),
        ],
        out_shape=[
            jax.ShapeDtypeStruct((B, D, T), jnp.float32),
            jax.ShapeDtypeStruct((1, 1), jnp.float32),
            jax.ShapeDtypeStruct((1, 1), jnp.float32),
        ],
        scratch_shapes=[
            pltpu.SMEM((1, 1), jnp.float32),
            pltpu.VMEM((K, 1), jnp.float32),
        ],
        compiler_params=pltpu.CompilerParams(
            dimension_semantics=("arbitrary",)),
    )(inputs, w, idx)


def _sc_scatter_body(idx_hbm, zeros_hbm, out_hbm, idx_v, rows_v):
    wid = lax.axis_index("s") * 2 + lax.axis_index("c")
    base = wid * ROWS_PER_WORKER

    pltpu.sync_copy(idx_hbm.at[pl.ds(base, ROWS_PER_WORKER)], idx_v)
    pltpu.sync_copy(zeros_hbm, rows_v)

    ones16 = jnp.full((16,), 1.0, jnp.float32)
    zero16 = jnp.zeros((16,), jnp.float32)
    lane = lax.iota(jnp.int32, 16)

    def chunk_body(ci, carry):
        for g in range(CHUNK // 16):
            cols = idx_v[pl.ds(ci * CHUNK + g * 16, 16)]
            rows = lane + g * 16
            plsc.store_scatter(rows_v, [rows, cols], ones16)
        pltpu.sync_copy(rows_v, out_hbm.at[pl.ds(base + ci * CHUNK, CHUNK)])
        for g in range(CHUNK // 16):
            cols = idx_v[pl.ds(ci * CHUNK + g * 16, 16)]
            rows = lane + g * 16
            plsc.store_scatter(rows_v, [rows, cols], zero16)
        return carry

    lax.fori_loop(0, ROWS_PER_WORKER // CHUNK, chunk_body, 0)


def _sc_scatter(idx_flat, zeros_chunk):
    mesh = plsc.VectorSubcoreMesh(core_axis_name="c", subcore_axis_name="s")
    f = pl.kernel(
        _sc_scatter_body,
        out_type=jax.ShapeDtypeStruct((N, K), jnp.float32),
        mesh=mesh,
        scratch_types=[
            pltpu.VMEM((ROWS_PER_WORKER,), jnp.int32),
            pltpu.VMEM((CHUNK, K), jnp.float32),
        ],
        compiler_params=pltpu.CompilerParams(needs_layout_passes=False),
    )
    return f(idx_flat, zeros_chunk)


@jax.jit
def kernel(inputs, W):
    idx = _argmin_call(inputs, W)

    zeros_chunk = jnp.zeros((CHUNK, K), jnp.float32)
    enc = _sc_scatter(idx.reshape(N), zeros_chunk)

    qst, loss, perp = _outputs_call(inputs, W, idx)

    return (loss.reshape(()), qst, perp.reshape(()), enc)


# speedup vs baseline: 1.4571x; 1.0356x over previous
import jax
import jax.numpy as jnp
from jax import lax
from jax.experimental import pallas as pl
from jax.experimental.pallas import tpu as pltpu
from jax.experimental.pallas import tpu_sc as plsc

K = 1024
D = 64
BETA = 0.25
B = 16
T = 1024
N = B * T

NUM_WORKERS = 32
ROWS_PER_WORKER = N // NUM_WORKERS
CHUNK = 64


BSTEP = 2


def _argmin_body(x_ref, w_ref, idx_ref):
    b = pl.program_id(0)
    w = w_ref[...]
    bsq = jnp.sum(w * w, axis=1, keepdims=True)
    w2 = w + w
    iota_k = lax.broadcasted_iota(jnp.int32, (K, T), 0)
    for p in range(BSTEP):
        x = x_ref[p]
        a = jnp.sum(x * x, axis=0, keepdims=True)
        m2 = lax.dot_general(w2, x, (((1,), (0,)), ((), ())),
                             preferred_element_type=jnp.float32)
        dist = (a + bsq) - m2
        minv = jnp.min(dist, axis=0, keepdims=True)
        idx = jnp.min(jnp.where(dist <= minv, iota_k, K), axis=0,
                      keepdims=True)
        idx_ref[b * BSTEP + p] = idx


def _argmin_call(inputs, w):
    return pl.pallas_call(
        _argmin_body,
        grid=(B // BSTEP,),
        in_specs=[
            pl.BlockSpec((BSTEP, D, T), lambda b: (b, 0, 0)),
            pl.BlockSpec((K, D), lambda b: (0, 0)),
        ],
        out_specs=pl.BlockSpec((B, 1, T), lambda b: (0, 0, 0)),
        out_shape=jax.ShapeDtypeStruct((B, 1, T), jnp.int32),
        compiler_params=pltpu.CompilerParams(
            dimension_semantics=("arbitrary",)),
    )(inputs, w)


def _outputs_body(x_ref, w_ref, idx_ref, qst_ref, loss_ref, perp_ref,
                  lacc_ref, hist_ref):
    b = pl.program_id(0)

    @pl.when(b == 0)
    def _init():
        lacc_ref[0, 0] = 0.0
        hist_ref[...] = jnp.zeros_like(hist_ref)

    x = x_ref[0]
    w = w_ref[...]
    idx = idx_ref[b]

    iota_k = lax.broadcasted_iota(jnp.int32, (K, T), 0)
    enc_t = (iota_k == idx).astype(jnp.float32)
    hist_ref[...] += jnp.sum(enc_t, axis=1, keepdims=True)

    q = lax.dot_general(w, enc_t, (((0,), (0,)), ((), ())),
                        preferred_element_type=jnp.float32)
    diff = q - x
    qst_ref[0] = x + diff
    lacc_ref[0, 0] += jnp.sum(diff * diff)

    @pl.when(b == B - 1)
    def _fin():
        mean_sq = lacc_ref[0, 0] / (B * T * D)
        loss_ref[0, 0] = mean_sq + BETA * mean_sq
        avg = hist_ref[...] * (1.0 / N)
        ent = avg * jnp.log(avg + 1e-10)
        perp_ref[0, 0] = jnp.exp(-jnp.sum(ent))


def _outputs_call(inputs, w, idx):
    return pl.pallas_call(
        _outputs_body,
        grid=(B,),
        in_specs=[
            pl.BlockSpec((1, D, T), lambda b: (b, 0, 0)),
            pl.BlockSpec((K, D), lambda b: (0, 0)),
            pl.BlockSpec((B, 1, T), lambda b: (0, 0, 0)),
        ],
        out_specs=[
            pl.BlockSpec((1, D, T), lambda b: (b, 0, 0)),
            pl.BlockSpec(memory_space=pltpu.SMEM),
            pl.BlockSpec(memory_space=pltpu.SMEM),
        ],
        out_shape=[
            jax.ShapeDtypeStruct((B, D, T), jnp.float32),
            jax.ShapeDtypeStruct((1, 1), jnp.float32),
            jax.ShapeDtypeStruct((1, 1), jnp.float32),
        ],
        scratch_shapes=[
            pltpu.SMEM((1, 1), jnp.float32),
            pltpu.VMEM((K, 1), jnp.float32),
        ],
        compiler_params=pltpu.CompilerParams(
            dimension_semantics=("arbitrary",)),
    )(inputs, w, idx)


def _sc_scatter_body(idx_hbm, zeros_hbm, out_hbm, idx_v, rows_v):
    wid = lax.axis_index("s") * 2 + lax.axis_index("c")
    base = wid * ROWS_PER_WORKER

    pltpu.sync_copy(idx_hbm.at[pl.ds(base, ROWS_PER_WORKER)], idx_v)
    pltpu.sync_copy(zeros_hbm, rows_v)

    ones16 = jnp.full((16,), 1.0, jnp.float32)
    zero16 = jnp.zeros((16,), jnp.float32)
    lane = lax.iota(jnp.int32, 16)

    def chunk_body(ci, carry):
        for g in range(CHUNK // 16):
            cols = idx_v[pl.ds(ci * CHUNK + g * 16, 16)]
            rows = lane + g * 16
            plsc.store_scatter(rows_v, [rows, cols], ones16)
        pltpu.sync_copy(rows_v, out_hbm.at[pl.ds(base + ci * CHUNK, CHUNK)])
        for g in range(CHUNK // 16):
            cols = idx_v[pl.ds(ci * CHUNK + g * 16, 16)]
            rows = lane + g * 16
            plsc.store_scatter(rows_v, [rows, cols], zero16)
        return carry

    lax.fori_loop(0, ROWS_PER_WORKER // CHUNK, chunk_body, 0)


def _sc_scatter(idx_flat, zeros_chunk):
    mesh = plsc.VectorSubcoreMesh(core_axis_name="c", subcore_axis_name="s")
    f = pl.kernel(
        _sc_scatter_body,
        out_type=jax.ShapeDtypeStruct((N, K), jnp.float32),
        mesh=mesh,
        scratch_types=[
            pltpu.VMEM((ROWS_PER_WORKER,), jnp.int32),
            pltpu.VMEM((CHUNK, K), jnp.float32),
        ],
        compiler_params=pltpu.CompilerParams(needs_layout_passes=False),
    )
    return f(idx_flat, zeros_chunk)


@jax.jit
def kernel(inputs, W):
    idx = _argmin_call(inputs, W)

    zeros_chunk = jnp.zeros((CHUNK, K), jnp.float32)
    enc = _sc_scatter(idx.reshape(N), zeros_chunk)

    qst, loss, perp = _outputs_call(inputs, W, idx)

    return (loss.reshape(()), qst, perp.reshape(()), enc)
